# Initial kernel scaffold; baseline (speedup 1.0000x reference)
import jax
import jax.numpy as jnp
from jax.experimental import pallas as pl


def kernel(feats_0, feats_1, feats_2, relation_features_0, relation_features_1, relation_features_2, nei_index_0, nei_index_1, fc1_W_0, fc1_b_0, fc2_W_0, fc2_b_0, fc1_W_1, fc1_b_1, fc2_W_1, fc2_b_1, fc1_W_2, fc1_b_2, fc2_W_2, fc2_b_2, att_h_0, att_r_0, att_h_1, att_r_1, inter_fc_W, inter_fc_b, inter_att, predict2_W, predict2_b, predict_W, predict_b, alpha):
    raise NotImplementedError("write your pallas kernel here")


# speedup vs baseline: 1.8087x; 1.8087x over previous
import functools

import jax
import jax.numpy as jnp
from jax import lax
from jax.experimental import pallas as pl
from jax.experimental.pallas import tpu as pltpu
from jax.experimental.pallas import tpu_sc as plsc

N0 = 10000
NT = 20000
DIN = 512
H = 256
S = 8
OUT = 64

_info = plsc.get_sparse_core_info()
_NC, _NS = _info.num_cores, _info.num_subcores
_NW = _NC * _NS
CHUNK = 16
NCHUNKS = N0 // CHUNK
STEPS = -(-NCHUNKS // _NW)


def _elu(x):
    return jnp.where(x > 0, x, jnp.exp(x) - 1.0)


def _proj_target_body(alpha_ref, x_ref, rel_ref, w1_ref, b1_ref, w2_ref,
                      b2_ref, a1_ref, tgt_ref, q_ref):
    a = alpha_ref[0, 0]
    h = _elu(jnp.dot(x_ref[...], w1_ref[...],
                     preferred_element_type=jnp.float32) + b1_ref[...])
    r = _elu(jnp.dot(rel_ref[...], w2_ref[...],
                     preferred_element_type=jnp.float32) + b2_ref[...])
    t = a * h + (1.0 - a) * r
    tgt_ref[...] = t
    q_ref[...] = jnp.dot(t, a1_ref[...], preferred_element_type=jnp.float32)


def _proj_target(alpha, x, rel, w1, b1, w2, b2, a1):
    B = 1000
    grid = (N0 // B,)
    return pl.pallas_call(
        _proj_target_body,
        grid=grid,
        in_specs=[
            pl.BlockSpec((1, 1), lambda i: (0, 0), memory_space=pltpu.SMEM),
            pl.BlockSpec((B, DIN), lambda i: (i, 0)),
            pl.BlockSpec((B, DIN), lambda i: (i, 0)),
            pl.BlockSpec((DIN, H), lambda i: (0, 0)),
            pl.BlockSpec((1, H), lambda i: (0, 0)),
            pl.BlockSpec((DIN, H), lambda i: (0, 0)),
            pl.BlockSpec((1, H), lambda i: (0, 0)),
            pl.BlockSpec((H, 4), lambda i: (0, 0)),
        ],
        out_specs=[
            pl.BlockSpec((B, H), lambda i: (i, 0)),
            pl.BlockSpec((B, 4), lambda i: (i, 0)),
        ],
        out_shape=[
            jax.ShapeDtypeStruct((N0, H), jnp.float32),
            jax.ShapeDtypeStruct((N0, 4), jnp.float32),
        ],
    )(alpha, x, rel, w1, b1, w2, b2, a1)


def _proj_table_body(x_ref, rel_ref, w1_ref, b1_ref, w2_ref, b2_ref,
                     a2h_ref, a2r_ref, h_ref, r_ref, p_ref):
    h = _elu(jnp.dot(x_ref[...], w1_ref[...],
                     preferred_element_type=jnp.float32) + b1_ref[...])
    r = _elu(jnp.dot(rel_ref[...], w2_ref[...],
                     preferred_element_type=jnp.float32) + b2_ref[...])
    h_ref[...] = h
    r_ref[...] = r
    ph = jnp.dot(h, a2h_ref[...], preferred_element_type=jnp.float32)
    pr = jnp.dot(r, a2r_ref[...], preferred_element_type=jnp.float32)
    p_ref[...] = jnp.concatenate([ph, pr], axis=1)


def _proj_table(x, rel, w1, b1, w2, b2, a2h, a2r):
    B = 1000
    grid = (NT // B,)
    return pl.pallas_call(
        _proj_table_body,
        grid=grid,
        in_specs=[
            pl.BlockSpec((B, DIN), lambda i: (i, 0)),
            pl.BlockSpec((B, DIN), lambda i: (i, 0)),
            pl.BlockSpec((DIN, H), lambda i: (0, 0)),
            pl.BlockSpec((1, H), lambda i: (0, 0)),
            pl.BlockSpec((DIN, H), lambda i: (0, 0)),
            pl.BlockSpec((1, H), lambda i: (0, 0)),
            pl.BlockSpec((H, 1), lambda i: (0, 0)),
            pl.BlockSpec((H, 1), lambda i: (0, 0)),
        ],
        out_specs=[
            pl.BlockSpec((B, H), lambda i: (i, 0)),
            pl.BlockSpec((B, H), lambda i: (i, 0)),
            pl.BlockSpec((B, 2), lambda i: (i, 0)),
        ],
        out_shape=[
            jax.ShapeDtypeStruct((NT, H), jnp.float32),
            jax.ShapeDtypeStruct((NT, H), jnp.float32),
            jax.ShapeDtypeStruct((NT, 2), jnp.float32),
        ],
    )(x, rel, w1, b1, w2, b2, a2h, a2r)


def _intra_sc_body(nei0_hbm, nei1_hbm, t0_hbm, t1_hbm, t2_hbm, t3_hbm,
                   p_hbm, q_hbm, e_hbm,
                   p_v, nei_v, q_v, rows_v, w_v, out_v, sem):
    wid = lax.axis_index("s") * _NC + lax.axis_index("c")
    lane = lax.iota(jnp.int32, 16)
    tabs = (t0_hbm, t1_hbm, t2_hbm, t3_hbm)
    for j in range(4):
        tab = tabs[j]
        nei = nei0_hbm if j < 2 else nei1_hbm
        pltpu.sync_copy(p_hbm.at[j], p_v)

        def chunk_body(ci, _, j=j, tab=tab, nei=nei):
            c = ci * _NW + wid

            @pl.when(c < NCHUNKS)
            def _():
                pltpu.sync_copy(nei.at[pl.ds(c * CHUNK * S, CHUNK * S)], nei_v)
                pltpu.sync_copy(q_hbm.at[j, pl.ds(c * CHUNK, CHUNK)], q_v)
                pltpu.async_copy(tab.at[nei_v], rows_v, sem).wait()
                qv = q_v[...]
                svecs = []
                for k in range(S):
                    idx_k = plsc.load_gather(nei_v, [lane * S + k])
                    pg = plsc.load_gather(p_v, [idx_k])
                    sc = qv + pg
                    svecs.append(jnp.where(sc > 0, sc, 0.01 * sc))
                m = svecs[0]
                for k in range(1, S):
                    m = jnp.maximum(m, svecs[k])
                evecs = [jnp.exp(sv - m) for sv in svecs]
                den = evecs[0]
                for k in range(1, S):
                    den = den + evecs[k]
                inv = 1.0 / den
                for k in range(S):
                    w_v[pl.ds(k * CHUNK, CHUNK)] = evecs[k] * inv

                def node_body(n, _):
                    base = n * S
                    zi = jnp.zeros((16,), jnp.int32)
                    wbs = [plsc.load_gather(w_v, [zi + (k * CHUNK + n)])
                           for k in range(S)]
                    for d in range(H // 16):
                        acc = None
                        for k in range(S):
                            rowd = rows_v[base + k, pl.ds(d * 16, 16)]
                            term = wbs[k] * rowd
                            acc = term if acc is None else acc + term
                        out_v[n, pl.ds(d * 16, 16)] = jnp.where(
                            acc > 0, acc, jnp.exp(acc) - 1.0)
                    return 0

                lax.fori_loop(0, CHUNK, node_body, 0)
                pltpu.sync_copy(out_v, e_hbm.at[j, pl.ds(c * CHUNK, CHUNK), :])

            return 0

        lax.fori_loop(0, STEPS, chunk_body, 0)


def _intra_sc(nei0, nei1, t0, t1, t2, t3, p_all, q_all):
    mesh = plsc.VectorSubcoreMesh(core_axis_name="c", subcore_axis_name="s")
    f = functools.partial(
        pl.kernel, mesh=mesh,
        compiler_params=pltpu.CompilerParams(needs_layout_passes=False),
        out_type=jax.ShapeDtypeStruct((4, N0, H), jnp.float32),
        scratch_types=[
            pltpu.VMEM((NT,), jnp.float32),
            pltpu.VMEM((CHUNK * S,), jnp.int32),
            pltpu.VMEM((CHUNK,), jnp.float32),
            pltpu.VMEM((CHUNK * S, H), jnp.float32),
            pltpu.VMEM((S * CHUNK,), jnp.float32),
            pltpu.VMEM((CHUNK, H), jnp.float32),
            pltpu.SemaphoreType.DMA,
        ],
    )(_intra_sc_body)
    return f(nei0, nei1, t0, t1, t2, t3, p_all, q_all)


def _beta_body(e_ref, w_ref, b_ref, s_ref):
    sums = []
    for j in range(4):
        t = jnp.tanh(jnp.dot(e_ref[j], w_ref[...],
                             preferred_element_type=jnp.float32) + b_ref[...])
        sums.append(jnp.sum(t, axis=0, keepdims=True))
    colsum = jnp.concatenate(sums, axis=0)

    @pl.when(pl.program_id(0) == 0)
    def _():
        s_ref[...] = jnp.zeros_like(s_ref)

    s_ref[...] += colsum


def _beta_sums(e_all, w, b):
    B = 1000
    grid = (N0 // B,)
    return pl.pallas_call(
        _beta_body,
        grid=grid,
        in_specs=[
            pl.BlockSpec((4, B, H), lambda i: (0, i, 0)),
            pl.BlockSpec((H, H), lambda i: (0, 0)),
            pl.BlockSpec((1, H), lambda i: (0, 0)),
        ],
        out_specs=pl.BlockSpec((4, H), lambda i: (0, 0)),
        out_shape=jax.ShapeDtypeStruct((4, H), jnp.float32),
    )(e_all, w, b)


def _combine_body(s_ref, att_ref, tgt_ref, e_ref, w2a_ref, w2b_ref, b2_ref,
                  wp_ref, bp_ref, h_ref, o_ref):
    l = jnp.sum(s_ref[...] * att_ref[...] * (1.0 / N0), axis=1, keepdims=True)
    m = jnp.max(l)
    ex = jnp.exp(l - m)
    beta = ex / jnp.sum(ex)
    e = e_ref[...]
    z = beta[0, 0] * e[0]
    for j in range(1, 4):
        z = z + beta[j, 0] * e[j]
    h = (jnp.dot(tgt_ref[...], w2a_ref[...],
                 preferred_element_type=jnp.float32)
         + jnp.dot(z, w2b_ref[...], preferred_element_type=jnp.float32)
         + b2_ref[...])
    h_ref[...] = h
    o_ref[...] = jnp.dot(h, wp_ref[...],
                         preferred_element_type=jnp.float32) + bp_ref[...]


def _combine(sums, att, tgt, e_all, w2a, w2b, b2, wp, bp):
    B = 1000
    grid = (N0 // B,)
    return pl.pallas_call(
        _combine_body,
        grid=grid,
        in_specs=[
            pl.BlockSpec((4, H), lambda i: (0, 0)),
            pl.BlockSpec((1, H), lambda i: (0, 0)),
            pl.BlockSpec((B, H), lambda i: (i, 0)),
            pl.BlockSpec((4, B, H), lambda i: (0, i, 0)),
            pl.BlockSpec((H, H), lambda i: (0, 0)),
            pl.BlockSpec((H, H), lambda i: (0, 0)),
            pl.BlockSpec((1, H), lambda i: (0, 0)),
            pl.BlockSpec((H, OUT), lambda i: (0, 0)),
            pl.BlockSpec((1, OUT), lambda i: (0, 0)),
        ],
        out_specs=[
            pl.BlockSpec((B, H), lambda i: (i, 0)),
            pl.BlockSpec((B, OUT), lambda i: (i, 0)),
        ],
        out_shape=[
            jax.ShapeDtypeStruct((N0, H), jnp.float32),
            jax.ShapeDtypeStruct((N0, OUT), jnp.float32),
        ],
    )(sums, att, tgt, e_all, w2a, w2b, b2, wp, bp)


def kernel(feats_0, feats_1, feats_2, relation_features_0,
           relation_features_1, relation_features_2, nei_index_0, nei_index_1,
           fc1_W_0, fc1_b_0, fc2_W_0, fc2_b_0, fc1_W_1, fc1_b_1, fc2_W_1,
           fc2_b_1, fc1_W_2, fc1_b_2, fc2_W_2, fc2_b_2, att_h_0, att_r_0,
           att_h_1, att_r_1, inter_fc_W, inter_fc_b, inter_att, predict2_W,
           predict2_b, predict_W, predict_b, alpha):
    r2 = lambda v: v.reshape(1, -1)
    a1 = jnp.stack([att_h_0[:H], att_r_0[:H], att_h_1[:H], att_r_1[:H]],
                   axis=1)
    tgt, q = _proj_target(jnp.reshape(alpha, (1, 1)), feats_0,
                          relation_features_0, fc1_W_0, r2(fc1_b_0),
                          fc2_W_0, r2(fc2_b_0), a1)
    h1, r1, p1 = _proj_table(feats_1, relation_features_1, fc1_W_1,
                             r2(fc1_b_1), fc2_W_1, r2(fc2_b_1),
                             att_h_0[H:].reshape(H, 1),
                             att_r_0[H:].reshape(H, 1))
    h2, r2t, p2 = _proj_table(feats_2, relation_features_2, fc1_W_2,
                              r2(fc1_b_2), fc2_W_2, r2(fc2_b_2),
                              att_h_1[H:].reshape(H, 1),
                              att_r_1[H:].reshape(H, 1))
    p_all = jnp.concatenate([p1.T, p2.T], axis=0)
    q_all = q.T
    e_all = _intra_sc(nei_index_0.reshape(-1), nei_index_1.reshape(-1),
                      h1, r1, h2, r2t, p_all, q_all)
    sums = _beta_sums(e_all, inter_fc_W, r2(inter_fc_b))
    h_out, out = _combine(sums, r2(inter_att), tgt, e_all,
                          predict2_W[:H], predict2_W[H:], r2(predict2_b),
                          predict_W, r2(predict_b))
    return out, h_out
